# baseline (speedup 1.0000x reference)
import functools

import jax
import jax.numpy as jnp
from jax import lax
from jax.experimental import pallas as pl
from jax.experimental.pallas import tpu as pltpu
from jax.experimental.pallas import tpu_sc as plsc

_GRP = 128
_K = 8
_BLK = _GRP * _K


def _sc_gather(table, idx3):
    NW, G, GRP = idx3.shape
    N, D = table.shape
    nblk = G // _K
    per_w = G * GRP
    out_rows = NW * per_w // 4
    mesh = plsc.VectorSubcoreMesh(core_axis_name="c", subcore_axis_name="s")
    NC = mesh.num_cores

    @functools.partial(
        pl.kernel,
        out_type=jax.ShapeDtypeStruct((out_rows, 4 * D), jnp.bfloat16),
        mesh=mesh,
        compiler_params=pltpu.CompilerParams(use_tc_tiling_on_sc=False),
        scratch_types=[
            pltpu.VMEM((G, GRP), jnp.int32),
            pltpu.VMEM((2, _BLK, D), jnp.bfloat16),
            pltpu.SemaphoreType.DMA,
            pltpu.SemaphoreType.DMA,
            pltpu.SemaphoreType.DMA,
            pltpu.SemaphoreType.DMA,
        ],
    )
    def gather_k(table_hbm, idx_hbm, out_hbm, idx_v, rows2, g0s, g1s, w0s, w1s):
        gsem = (g0s, g1s)
        wsem = (w0s, w1s)
        wid = lax.axis_index("s") * NC + lax.axis_index("c")
        pltpu.sync_copy(idx_hbm.at[wid], idx_v)
        r = wid // 2
        q = r // 4
        t = r % 4
        row0 = q * N + (wid % 2) * per_w

        def out_slice(blk):
            return out_hbm.at[pl.ds(row0 + blk * _BLK, _BLK), pl.ds(t * D, D)]

        def fire(blk, p):
            for b in range(_K):
                pltpu.async_copy(
                    table_hbm.at[idx_v.at[blk * _K + b]],
                    rows2.at[p].at[pl.ds(b * GRP, GRP)],
                    gsem[p],
                )

        def drain(blk, p):
            for b in range(_K):
                pltpu.make_async_copy(
                    table_hbm.at[idx_v.at[blk * _K + b]],
                    rows2.at[p].at[pl.ds(b * GRP, GRP)],
                    gsem[p],
                ).wait()

        fire(0, 0)

        @pl.loop(0, nblk // 2)
        def _pair(i):
            for p in (0, 1):
                j = 2 * i + p
                o = 1 - p

                def _fire_next(j=j, p=p, o=o):
                    @pl.when(j >= 1)
                    def _():
                        pltpu.make_async_copy(
                            rows2.at[o], out_slice(j - 1), wsem[o]
                        ).wait()
                    fire(j + 1, o)

                if p == 0:
                    _fire_next()
                else:
                    pl.when(i < nblk // 2 - 1)(_fire_next)

                drain(j, p)
                pltpu.async_copy(rows2.at[p], out_slice(j), wsem[p])

        pltpu.make_async_copy(rows2.at[0], out_slice(nblk - 2), w0s).wait()
        pltpu.make_async_copy(rows2.at[1], out_slice(nblk - 1), w1s).wait()

    return gather_k(table, idx3)


def _prep_body(x_ref, o_ref):
    xb = x_ref[...]
    bt, c, bx, y, z = xb.shape
    o_ref[...] = xb.reshape(bt * c, bx * y * z).T.astype(jnp.bfloat16)


def _prep_table(x):
    BT, C, X, Y, Z = x.shape
    BX = 2
    BN = BX * Y * Z
    N = X * Y * Z
    return pl.pallas_call(
        _prep_body,
        grid=(X // BX,),
        in_specs=[pl.BlockSpec((BT, C, BX, Y, Z), lambda i: (0, 0, i, 0, 0))],
        out_specs=pl.BlockSpec((BN, BT * C), lambda i: (i, 0)),
        out_shape=jax.ShapeDtypeStruct((N, BT * C), jnp.bfloat16),
    )(x)


def _mix_body(ga, gb, gc, gd, w_ref, b_ref, o_ref):
    acc = b_ref[...]
    for q, g_ref in enumerate((ga, gb, gc, gd)):
        acc = acc + lax.dot_general(
            w_ref[q],
            g_ref[...],
            (((0,), (1,)), ((), ())),
            preferred_element_type=jnp.float32,
        )
    bt, c, bx, y, z = o_ref.shape
    o_ref[...] = acc.reshape(bt, c, bx, y, z)


def _mix_matmul(g3, w3, b2, grid_shape):
    BT, O, X, Y, Z = grid_shape
    BX = 2
    BM = BX * Y * Z
    n_nodes = X * Y * Z
    nb = n_nodes // BM
    g_spec = lambda q: pl.BlockSpec((BM, 128), lambda i, q=q: (q * nb + i, 0))
    return pl.pallas_call(
        _mix_body,
        grid=(nb,),
        in_specs=[
            g_spec(0), g_spec(1), g_spec(2), g_spec(3),
            pl.BlockSpec((4, 128, 32), lambda i: (0, 0, 0)),
            pl.BlockSpec((32, 1), lambda i: (0, 0)),
        ],
        out_specs=pl.BlockSpec((BT, O, BX, Y, Z), lambda i: (0, 0, i, 0, 0)),
        out_shape=jax.ShapeDtypeStruct((BT, O, X, Y, Z), jnp.float32),
    )(g3, g3, g3, g3, w3, b2)


def kernel(x, rand_indices, weight, bias):
    BT, C, X, Y, Z = x.shape
    N = X * Y * Z
    R = rand_indices.shape[1]
    O = weight.shape[0]
    D = BT * C
    NW = 32
    assert (N * R) % (NW * _GRP) == 0 and R == 16 and D == 32

    x_t = _prep_table(x)
    idx3 = rand_indices.T.reshape(NW, (N * R) // (NW * _GRP), _GRP)
    g3 = _sc_gather(x_t, idx3)

    wt = jnp.transpose(weight, (2, 1, 0))
    eye = jnp.eye(BT, dtype=weight.dtype)
    w2 = jnp.einsum("rco,bd->rbcdo", wt, eye).reshape(R * D, BT * O)
    w3 = w2.reshape(4, 128, BT * O).astype(jnp.bfloat16)
    b2 = jnp.tile(bias, BT).reshape(BT * O, 1)

    return _mix_matmul(g3, w3, b2, (BT, O, X, Y, Z))


# speedup vs baseline: 2.3904x; 2.3904x over previous
import functools

import jax
import jax.numpy as jnp
from jax import lax
from jax.experimental import pallas as pl
from jax.experimental.pallas import tpu as pltpu
from jax.experimental.pallas import tpu_sc as plsc

_GRP = 128
_K = 8
_BLK = _GRP * _K


def _sc_gather(table, idx3):
    NW, G, GRP = idx3.shape
    N, D = table.shape
    nblk = G // _K
    per_w = G * GRP
    out_rows = NW * per_w // 4
    mesh = plsc.VectorSubcoreMesh(core_axis_name="c", subcore_axis_name="s")
    NC = mesh.num_cores

    @functools.partial(
        pl.kernel,
        out_type=jax.ShapeDtypeStruct((out_rows, 4 * D), jnp.float32),
        mesh=mesh,
        compiler_params=pltpu.CompilerParams(use_tc_tiling_on_sc=False),
        scratch_types=[
            pltpu.VMEM((G, GRP), jnp.int32),
            pltpu.VMEM((2, _BLK, D), jnp.float32),
            pltpu.SemaphoreType.DMA,
            pltpu.SemaphoreType.DMA,
            pltpu.SemaphoreType.DMA,
            pltpu.SemaphoreType.DMA,
        ],
    )
    def gather_k(table_hbm, idx_hbm, out_hbm, idx_v, rows2, g0s, g1s, w0s, w1s):
        gsem = (g0s, g1s)
        wsem = (w0s, w1s)
        wid = lax.axis_index("s") * NC + lax.axis_index("c")
        pltpu.sync_copy(idx_hbm.at[wid], idx_v)
        r = wid // 2
        q = r // 4
        t = r % 4
        row0 = q * N + (wid % 2) * per_w

        def out_slice(blk):
            return out_hbm.at[pl.ds(row0 + blk * _BLK, _BLK), pl.ds(t * D, D)]

        def fire(blk, p):
            for b in range(_K):
                pltpu.async_copy(
                    table_hbm.at[idx_v.at[blk * _K + b]],
                    rows2.at[p].at[pl.ds(b * GRP, GRP)],
                    gsem[p],
                )

        def drain(blk, p):
            for b in range(_K):
                pltpu.make_async_copy(
                    table_hbm.at[idx_v.at[blk * _K + b]],
                    rows2.at[p].at[pl.ds(b * GRP, GRP)],
                    gsem[p],
                ).wait()

        fire(0, 0)

        @pl.loop(0, nblk // 2)
        def _pair(i):
            for p in (0, 1):
                j = 2 * i + p
                o = 1 - p

                def _fire_next(j=j, p=p, o=o):
                    @pl.when(j >= 1)
                    def _():
                        pltpu.make_async_copy(
                            rows2.at[o], out_slice(j - 1), wsem[o]
                        ).wait()
                    fire(j + 1, o)

                if p == 0:
                    _fire_next()
                else:
                    pl.when(i < nblk // 2 - 1)(_fire_next)

                drain(j, p)
                pltpu.async_copy(rows2.at[p], out_slice(j), wsem[p])

        pltpu.make_async_copy(rows2.at[0], out_slice(nblk - 2), w0s).wait()
        pltpu.make_async_copy(rows2.at[1], out_slice(nblk - 1), w1s).wait()

    return gather_k(table, idx3)


def _prep_body(x_ref, o_ref):
    xb = x_ref[...]
    bt, c, bx, y, z = xb.shape
    o_ref[...] = xb.reshape(bt * c, bx * y * z).T


def _prep_table(x):
    BT, C, X, Y, Z = x.shape
    BX = 2
    BN = BX * Y * Z
    N = X * Y * Z
    return pl.pallas_call(
        _prep_body,
        grid=(X // BX,),
        in_specs=[pl.BlockSpec((BT, C, BX, Y, Z), lambda i: (0, 0, i, 0, 0))],
        out_specs=pl.BlockSpec((BN, BT * C), lambda i: (i, 0)),
        out_shape=jax.ShapeDtypeStruct((N, BT * C), jnp.float32),
    )(x)


def _mix_body(ga, gb, gc, gd, w_ref, b_ref, o_ref):
    acc = b_ref[...]
    for q, g_ref in enumerate((ga, gb, gc, gd)):
        acc = acc + lax.dot_general(
            w_ref[q],
            g_ref[...],
            (((0,), (1,)), ((), ())),
            preferred_element_type=jnp.float32,
        )
    bt, c, bx, y, z = o_ref.shape
    o_ref[...] = acc.reshape(bt, c, bx, y, z)


def _mix_matmul(g3, w3, b2, grid_shape):
    BT, O, X, Y, Z = grid_shape
    BX = 2
    BM = BX * Y * Z
    n_nodes = X * Y * Z
    nb = n_nodes // BM
    g_spec = lambda q: pl.BlockSpec((BM, 128), lambda i, q=q: (q * nb + i, 0))
    return pl.pallas_call(
        _mix_body,
        grid=(nb,),
        in_specs=[
            g_spec(0), g_spec(1), g_spec(2), g_spec(3),
            pl.BlockSpec((4, 128, 32), lambda i: (0, 0, 0)),
            pl.BlockSpec((32, 1), lambda i: (0, 0)),
        ],
        out_specs=pl.BlockSpec((BT, O, BX, Y, Z), lambda i: (0, 0, i, 0, 0)),
        out_shape=jax.ShapeDtypeStruct((BT, O, X, Y, Z), jnp.float32),
    )(g3, g3, g3, g3, w3, b2)


def kernel(x, rand_indices, weight, bias):
    BT, C, X, Y, Z = x.shape
    N = X * Y * Z
    R = rand_indices.shape[1]
    O = weight.shape[0]
    D = BT * C
    NW = 32
    assert (N * R) % (NW * _GRP) == 0 and R == 16 and D == 32

    x_p = jnp.transpose(x, (0, 1, 2, 4, 3))
    x_t = _prep_table(x_p)
    yz = Y * Z
    ix = rand_indices // yz
    rem = rand_indices % yz
    riphi = ix * yz + (rem % Z) * Y + rem // Z
    idx3 = riphi.T.reshape(NW, (N * R) // (NW * _GRP), _GRP)
    g3 = _sc_gather(x_t, idx3)

    wt = jnp.transpose(weight, (2, 1, 0))
    eye = jnp.eye(BT, dtype=weight.dtype)
    w2 = jnp.einsum("rco,bd->rbcdo", wt, eye).reshape(R * D, BT * O)
    w3 = w2.reshape(4, 128, BT * O)
    b2 = jnp.tile(bias, BT).reshape(BT * O, 1)

    out_p = _mix_matmul(g3, w3, b2, (BT, O, X, Z, Y))
    return jnp.transpose(out_p, (0, 1, 2, 4, 3))
